# baseline (speedup 1.0000x reference)
import jax
import jax.numpy as jnp
from jax.experimental import pallas as pl
from jax.experimental.pallas import tpu as pltpu

_S = 64
_H = 512
_I = 512
_E = 16
_G = 8
_C = _I // 2
_ALPHA = 1.702
_LIMIT = 7.0
_EPS = 1e-5


def _moe_kernel(x_ref, scale_ref, gate_ref, w1a_ref, w1b_ref, b1_ref,
                w2a_ref, w2b_ref, b2_ref, out_ref, t_ref, rw_ref, p_ref):
    e = pl.program_id(0)

    @pl.when(e == 0)
    def _prologue():
        x = x_ref[...]
        v = jnp.mean(x * x, axis=-1, keepdims=True)
        t = x * jax.lax.rsqrt(v + _EPS) * scale_ref[...]
        t_ref[...] = t
        g = jax.lax.dot_general(t, gate_ref[...], (((1,), (1,)), ((), ())),
                                preferred_element_type=jnp.float32)
        iota = jax.lax.broadcasted_iota(jnp.int32, (_S, _E), 1)
        v1 = jnp.max(g, axis=1, keepdims=True)
        i1 = jnp.min(jnp.where(g == v1, iota, _E), axis=1, keepdims=True)
        m1 = iota == i1
        gm = jnp.where(m1, -jnp.inf, g)
        v2 = jnp.max(gm, axis=1, keepdims=True)
        i2 = jnp.min(jnp.where(gm == v2, iota, _E), axis=1, keepdims=True)
        m2 = iota == i2
        p1 = jax.nn.sigmoid(v1 - v2)
        rw_ref[...] = jnp.where(m1, p1, 0.0) + jnp.where(m2, 1.0 - p1, 0.0)
        r = jax.lax.broadcasted_iota(jnp.int32, (_I, _C), 0)
        c = jax.lax.broadcasted_iota(jnp.int32, (_I, _C), 1)
        p_ref[...] = (r == 2 * c).astype(jnp.float32)
        out_ref[...] = x

    t = t_ref[...]
    iota = jax.lax.broadcasted_iota(jnp.int32, (_S, _E), 1)
    rw = rw_ref[...]
    acc = out_ref[...]
    lane = jax.lax.broadcasted_iota(jnp.int32, (_S, _I), 1)
    even = (lane % 2) == 0
    p = p_ref[...]

    def half(w1_half, b1_half, w2_half):
        h = jax.lax.dot_general(t, w1_half, (((1,), (1,)), ((), ())),
                                preferred_element_type=jnp.float32) + b1_half
        a = jnp.minimum(h, _LIMIT)
        a = a * jax.nn.sigmoid(_ALPHA * a)
        b = jnp.clip(h, -_LIMIT, _LIMIT) + 1.0
        act_z = jnp.where(even, a * jnp.roll(b, -1, axis=1), 0.0)
        act = jax.lax.dot_general(act_z, p, (((1,), (0,)), ((), ())),
                                  preferred_element_type=jnp.float32)
        return jax.lax.dot_general(act, w2_half, (((1,), (1,)), ((), ())),
                                   preferred_element_type=jnp.float32)

    for j in range(_G):
        b1 = b1_ref[j]
        o = (half(w1a_ref[j, 0], b1[:, :_I], w2a_ref[j])
             + half(w1b_ref[j, 0], b1[:, _I:], w2b_ref[j])
             + b2_ref[j])
        w_col = jnp.sum(jnp.where(iota == e * _G + j, rw, 0.0), axis=1,
                        keepdims=True)
        acc = acc + o * w_col
    out_ref[...] = acc


def kernel(x, norm_scale, gate_w, mlp1_w, mlp1_b, mlp2_w, mlp2_b):
    w1v = mlp1_w.reshape(_E, 2, _I, _H)
    b1v = mlp1_b.reshape(_E, 1, 2 * _I)
    b2v = mlp2_b.reshape(_E, 1, _H)
    scale2d = norm_scale.reshape(1, _H)

    in_specs = [
            pl.BlockSpec((_S, _H), lambda e: (0, 0)),
            pl.BlockSpec((1, _H), lambda e: (0, 0)),
            pl.BlockSpec((_E, _H), lambda e: (0, 0)),
            pl.BlockSpec((_G, 1, _I, _H), lambda e: (e, 0, 0, 0)),
            pl.BlockSpec((_G, 1, _I, _H), lambda e: (e, 1, 0, 0)),
            pl.BlockSpec((_G, 1, 2 * _I), lambda e: (e, 0, 0)),
            pl.BlockSpec((_G, _H, _C), lambda e: (e, 0, 0)),
            pl.BlockSpec((_G, _H, _C), lambda e: (e, 0, 1)),
            pl.BlockSpec((_G, 1, _H), lambda e: (e, 0, 0)),
    ]
    return pl.pallas_call(
        _moe_kernel,
        grid=(_E // _G,),
        in_specs=in_specs,
        out_specs=pl.BlockSpec((_S, _H), lambda e: (0, 0)),
        out_shape=jax.ShapeDtypeStruct((_S, _H), jnp.float32),
        scratch_shapes=[
            pltpu.VMEM((_S, _H), jnp.float32),
            pltpu.VMEM((_S, _E), jnp.float32),
            pltpu.VMEM((_I, _C), jnp.float32),
        ],
        compiler_params=pltpu.CompilerParams(
            dimension_semantics=("arbitrary",),
        ),
    )(x, scale2d, gate_w, w1v, w1v, b1v, mlp2_w, mlp2_w, b2v)


# speedup vs baseline: 1.0611x; 1.0611x over previous
import jax
import jax.numpy as jnp
from jax.experimental import pallas as pl
from jax.experimental.pallas import tpu as pltpu

_S = 64
_H = 512
_I = 512
_E = 16
_G = 4
_ALPHA = 1.702
_LIMIT = 7.0
_EPS = 1e-5


def _moe_kernel(x_ref, scale_ref, gate_ref, w1_ref, b1_ref,
                w2_ref, b2_ref, out_ref, t_ref, rw_ref, p_ref):
    e = pl.program_id(0)

    @pl.when(e == 0)
    def _prologue():
        x = x_ref[...]
        v = jnp.mean(x * x, axis=-1, keepdims=True)
        t = x * jax.lax.rsqrt(v + _EPS) * scale_ref[...]
        t_ref[...] = t
        g = jax.lax.dot_general(t, gate_ref[...], (((1,), (1,)), ((), ())),
                                preferred_element_type=jnp.float32)
        iota = jax.lax.broadcasted_iota(jnp.int32, (_S, _E), 1)
        v1 = jnp.max(g, axis=1, keepdims=True)
        i1 = jnp.min(jnp.where(g == v1, iota, _E), axis=1, keepdims=True)
        m1 = iota == i1
        gm = jnp.where(m1, -jnp.inf, g)
        v2 = jnp.max(gm, axis=1, keepdims=True)
        i2 = jnp.min(jnp.where(gm == v2, iota, _E), axis=1, keepdims=True)
        m2 = iota == i2
        p1 = jax.nn.sigmoid(v1 - v2)
        rw_ref[...] = jnp.where(m1, p1, 0.0) + jnp.where(m2, 1.0 - p1, 0.0)
        r = jax.lax.broadcasted_iota(jnp.int32, (2 * _I, _I), 0)
        c = jax.lax.broadcasted_iota(jnp.int32, (2 * _I, _I), 1)
        p_ref[...] = (r == 2 * c).astype(jnp.float32)
        out_ref[...] = x

    t = t_ref[...]
    iota = jax.lax.broadcasted_iota(jnp.int32, (_S, _E), 1)
    rw = rw_ref[...]
    acc = out_ref[...]
    lane = jax.lax.broadcasted_iota(jnp.int32, (_S, _G * 2 * _I), 1)
    even = (lane % 2) == 0
    p = p_ref[...]

    h = jax.lax.dot_general(t, w1_ref[...], (((1,), (1,)), ((), ())),
                            preferred_element_type=jnp.float32) + b1_ref[0]
    a = jnp.minimum(h, _LIMIT)
    a = a * jax.nn.sigmoid(_ALPHA * a)
    b = jnp.clip(h, -_LIMIT, _LIMIT) + 1.0
    act_z = jnp.where(even, a * jnp.roll(b, -1, axis=1), 0.0)

    for j in range(_G):
        az = act_z[:, j * 2 * _I:(j + 1) * 2 * _I]
        act = jax.lax.dot_general(az, p, (((1,), (0,)), ((), ())),
                                  preferred_element_type=jnp.float32)
        o = jax.lax.dot_general(act, w2_ref[j], (((1,), (1,)), ((), ())),
                                preferred_element_type=jnp.float32) + b2_ref[j]
        w_col = jnp.sum(jnp.where(iota == e * _G + j, rw, 0.0), axis=1,
                        keepdims=True)
        acc = acc + o * w_col
    out_ref[...] = acc


def kernel(x, norm_scale, gate_w, mlp1_w, mlp1_b, mlp2_w, mlp2_b):
    w1v = mlp1_w.reshape(_E * 2 * _I, _H)
    b1v = mlp1_b.reshape(_E // _G, 1, _G * 2 * _I)
    b2v = mlp2_b.reshape(_E, 1, _H)
    scale2d = norm_scale.reshape(1, _H)

    in_specs = [
            pl.BlockSpec((_S, _H), lambda e: (0, 0)),
            pl.BlockSpec((1, _H), lambda e: (0, 0)),
            pl.BlockSpec((_E, _H), lambda e: (0, 0)),
            pl.BlockSpec((_G * 2 * _I, _H), lambda e: (e, 0)),
            pl.BlockSpec((1, 1, _G * 2 * _I), lambda e: (e, 0, 0)),
            pl.BlockSpec((_G, _H, _I), lambda e: (e, 0, 0)),
            pl.BlockSpec((_G, 1, _H), lambda e: (e, 0, 0)),
    ]
    return pl.pallas_call(
        _moe_kernel,
        grid=(_E // _G,),
        in_specs=in_specs,
        out_specs=pl.BlockSpec((_S, _H), lambda e: (0, 0)),
        out_shape=jax.ShapeDtypeStruct((_S, _H), jnp.float32),
        scratch_shapes=[
            pltpu.VMEM((_S, _H), jnp.float32),
            pltpu.VMEM((_S, _E), jnp.float32),
            pltpu.VMEM((2 * _I, _I), jnp.float32),
        ],
        compiler_params=pltpu.CompilerParams(
            dimension_semantics=("arbitrary",),
        ),
    )(x, scale2d, gate_w, w1v, b1v, mlp2_w, b2v)


# speedup vs baseline: 1.1015x; 1.0381x over previous
import jax
import jax.numpy as jnp
from jax.experimental import pallas as pl
from jax.experimental.pallas import tpu as pltpu

_S = 64
_H = 512
_I = 512
_E = 16
_G = 2
_ALPHA = 1.702
_LIMIT = 7.0
_EPS = 1e-5


def _moe_kernel(x_ref, scale_ref, gate_ref, w1_ref, b1_ref,
                w2_ref, b2_ref, out_ref, t_ref, rw_ref, p_ref):
    e = pl.program_id(0)

    @pl.when(e == 0)
    def _prologue():
        x = x_ref[...]
        v = jnp.mean(x * x, axis=-1, keepdims=True)
        t = x * jax.lax.rsqrt(v + _EPS) * scale_ref[...]
        t_ref[...] = t
        g = jax.lax.dot_general(t, gate_ref[...], (((1,), (1,)), ((), ())),
                                preferred_element_type=jnp.float32)
        iota = jax.lax.broadcasted_iota(jnp.int32, (_S, _E), 1)
        v1 = jnp.max(g, axis=1, keepdims=True)
        i1 = jnp.min(jnp.where(g == v1, iota, _E), axis=1, keepdims=True)
        m1 = iota == i1
        gm = jnp.where(m1, -jnp.inf, g)
        v2 = jnp.max(gm, axis=1, keepdims=True)
        i2 = jnp.min(jnp.where(gm == v2, iota, _E), axis=1, keepdims=True)
        m2 = iota == i2
        p1 = jax.nn.sigmoid(v1 - v2)
        rw_ref[...] = jnp.where(m1, p1, 0.0) + jnp.where(m2, 1.0 - p1, 0.0)
        r = jax.lax.broadcasted_iota(jnp.int32, (2 * _I, _I), 0)
        c = jax.lax.broadcasted_iota(jnp.int32, (2 * _I, _I), 1)
        p_ref[...] = (r == 2 * c).astype(jnp.float32)
        out_ref[...] = x

    t = t_ref[...]
    iota = jax.lax.broadcasted_iota(jnp.int32, (_S, _E), 1)
    rw = rw_ref[...]
    acc = out_ref[...]
    lane = jax.lax.broadcasted_iota(jnp.int32, (_S, _G * 2 * _I), 1)
    even = (lane % 2) == 0
    p = p_ref[...]

    h = jax.lax.dot_general(t, w1_ref[...], (((1,), (1,)), ((), ())),
                            preferred_element_type=jnp.float32) + b1_ref[0]
    a = jnp.minimum(h, _LIMIT)
    a = a * jax.nn.sigmoid(_ALPHA * a)
    b = jnp.clip(h, -_LIMIT, _LIMIT) + 1.0
    act_z = jnp.where(even, a * jnp.roll(b, -1, axis=1), 0.0)

    for j in range(_G):
        az = act_z[:, j * 2 * _I:(j + 1) * 2 * _I]
        act = jax.lax.dot_general(az, p, (((1,), (0,)), ((), ())),
                                  preferred_element_type=jnp.float32)
        o = jax.lax.dot_general(act, w2_ref[j], (((1,), (1,)), ((), ())),
                                preferred_element_type=jnp.float32) + b2_ref[j]
        w_col = jnp.sum(jnp.where(iota == e * _G + j, rw, 0.0), axis=1,
                        keepdims=True)
        acc = acc + o * w_col
    out_ref[...] = acc


def kernel(x, norm_scale, gate_w, mlp1_w, mlp1_b, mlp2_w, mlp2_b):
    w1v = mlp1_w.reshape(_E * 2 * _I, _H)
    b1v = mlp1_b.reshape(_E // _G, 1, _G * 2 * _I)
    b2v = mlp2_b.reshape(_E, 1, _H)
    scale2d = norm_scale.reshape(1, _H)

    in_specs = [
            pl.BlockSpec((_S, _H), lambda e: (0, 0)),
            pl.BlockSpec((1, _H), lambda e: (0, 0)),
            pl.BlockSpec((_E, _H), lambda e: (0, 0)),
            pl.BlockSpec((_G * 2 * _I, _H), lambda e: (e, 0)),
            pl.BlockSpec((1, 1, _G * 2 * _I), lambda e: (e, 0, 0)),
            pl.BlockSpec((_G, _H, _I), lambda e: (e, 0, 0)),
            pl.BlockSpec((_G, 1, _H), lambda e: (e, 0, 0)),
    ]
    return pl.pallas_call(
        _moe_kernel,
        grid=(_E // _G,),
        in_specs=in_specs,
        out_specs=pl.BlockSpec((_S, _H), lambda e: (0, 0)),
        out_shape=jax.ShapeDtypeStruct((_S, _H), jnp.float32),
        scratch_shapes=[
            pltpu.VMEM((_S, _H), jnp.float32),
            pltpu.VMEM((_S, _E), jnp.float32),
            pltpu.VMEM((2 * _I, _I), jnp.float32),
        ],
        compiler_params=pltpu.CompilerParams(
            dimension_semantics=("arbitrary",),
        ),
    )(x, scale2d, gate_w, w1v, b1v, mlp2_w, b2v)
